# Initial kernel scaffold; baseline (speedup 1.0000x reference)
import jax
import jax.numpy as jnp
from jax.experimental import pallas as pl


def kernel(x, edge_index):
    raise NotImplementedError("write your pallas kernel here")


# speedup vs baseline: 1.9488x; 1.9488x over previous
import functools

import jax
import jax.numpy as jnp
from jax import lax
from jax.experimental import pallas as pl
from jax.experimental.pallas import tpu as pltpu
from jax.experimental.pallas import tpu_sc as plsc

D = 128
NC = 2
NS = 16
NW = NC * NS


def _gather_body(rows, iters, x_hbm, idx_hbm, out_hbm, idx_v, rows_v, sem):
    wid = lax.axis_index("s") * NC + lax.axis_index("c")

    @pl.loop(0, iters)
    def _iter(j):
        r = j * NW + wid

        @pl.when(r < rows)
        def _():
            pltpu.sync_copy(idx_hbm.at[r], idx_v)
            pltpu.async_copy(x_hbm.at[idx_v], rows_v, sem).wait()
            pltpu.sync_copy(rows_v, out_hbm.at[pl.ds(r * D, D)])


@jax.jit
def kernel(x, edge_index):
    n_edges = edge_index.shape[1]
    rows = 2 * n_edges // D
    iters = -(-rows // NW)
    idx2 = jnp.transpose(edge_index).reshape(rows, D)
    mesh = plsc.VectorSubcoreMesh(
        core_axis_name="c", subcore_axis_name="s", num_cores=NC, num_subcores=NS
    )
    run = pl.kernel(
        functools.partial(_gather_body, rows, iters),
        out_type=jax.ShapeDtypeStruct((2 * n_edges, x.shape[1]), jnp.float32),
        mesh=mesh,
        scratch_types=[
            pltpu.VMEM((D,), jnp.int32),
            pltpu.VMEM((D, D), jnp.float32),
            pltpu.SemaphoreType.DMA,
        ],
    )
    out = run(x, idx2)
    return out.reshape(n_edges, 2 * x.shape[1])
